# baseline (speedup 1.0000x reference)
import functools

import numpy as np
import jax
import jax.numpy as jnp
from jax import lax
from jax.experimental import pallas as pl
from jax.experimental.pallas import tpu as pltpu
from jax.experimental.pallas import tpu_sc as plsc

_BINS = (512, 128, 256, 256, 64, 256, 256, 16, 256, 64, 16, 128, 64, 128, 64, 512, 512)
_EMB = 16
_NF = 17
_BATCH = 4096
_EPS = 1e-5
_OFFS = np.concatenate([[0], np.cumsum(_BINS)[:-1]]).astype(np.int32)
_VOCAB = int(np.sum(_BINS))

_NC, _NS = 2, 16
_NW = _NC * _NS
_B_TOT = _BATCH * _NF
_BPW = _B_TOT // _NW
_ROWS_PW = _BATCH // _NW
_QSTEPS = _BPW // 16


def _gather_body(*refs):
    embs = refs[:_NF]
    idx_hbm = refs[_NF]
    h_hbm = refs[_NF + 1]
    table_v, idx_v, out_v, sem = refs[_NF + 2:]

    wid = lax.axis_index("s") * _NC + lax.axis_index("c")

    copies = [
        pltpu.async_copy(
            embs[i], table_v.at[pl.ds(int(_OFFS[i]), _BINS[i]), :], sem)
        for i in range(_NF)
    ]
    pltpu.sync_copy(idx_hbm.at[:, pl.ds(wid * _ROWS_PW, _ROWS_PW)], idx_v)
    for c in copies:
        c.wait()

    iota = lax.iota(jnp.int32, 16)

    def step(g, _):
        row = iota + g * 16
        for i in range(_NF):
            idx = idx_v[i, pl.ds(g * 16, 16)]
            for c in range(_EMB):
                csplat = jnp.full((16,), c, jnp.int32)
                val = plsc.load_gather(table_v, [idx, csplat])
                col = jnp.full((16,), i * _EMB + c, jnp.int32)
                plsc.store_scatter(out_v, [row, col], val)
        return _

    lax.fori_loop(0, _ROWS_PW // 16, step, None)

    pltpu.sync_copy(out_v, h_hbm.at[pl.ds(wid * _ROWS_PW, _ROWS_PW), :])


@functools.lru_cache(maxsize=None)
def _sc_gather():
    return pl.kernel(
        _gather_body,
        out_type=jax.ShapeDtypeStruct((_BATCH, _NF * _EMB), jnp.float32),
        mesh=plsc.VectorSubcoreMesh(core_axis_name="c", subcore_axis_name="s",
                                    num_cores=_NC, num_subcores=_NS),
        scratch_types=[
            pltpu.VMEM((_VOCAB, _EMB), jnp.float32),
            pltpu.VMEM((_NF, _ROWS_PW), jnp.int32),
            pltpu.VMEM((_ROWS_PW, _NF * _EMB), jnp.float32),
            pltpu.SemaphoreType.DMA,
        ],
        compiler_params=pltpu.CompilerParams(use_tc_tiling_on_sc=False,
                                             needs_layout_passes=False),
    )


def _mlp_body(h_ref, w0, b0, g0, be0, w1, b1, g1, be1, w2, b2, g2, be2, w3, b3,
              out_ref):
    def layer(h, w, b, g, be):
        h = jnp.dot(h, w[:], preferred_element_type=jnp.float32) + b[:]
        h = jnp.maximum(h, 0.0)
        m = jnp.mean(h, axis=0, keepdims=True)
        v = jnp.mean((h - m) ** 2, axis=0, keepdims=True)
        return (h - m) * (g[:] * lax.rsqrt(v + _EPS)) + be[:]

    h = h_ref[:]
    h = layer(h, w0, b0, g0, be0)
    h = layer(h, w1, b1, g1, be1)
    h = layer(h, w2, b2, g2, be2)
    out_ref[:] = jnp.dot(h, w3[:], preferred_element_type=jnp.float32) + b3[:]


_mlp = pl.pallas_call(
    _mlp_body,
    out_shape=jax.ShapeDtypeStruct((_BATCH, 1), jnp.float32),
)


def kernel(x, emb_0, emb_1, emb_2, emb_3, emb_4, emb_5, emb_6, emb_7, emb_8,
           emb_9, emb_10, emb_11, emb_12, emb_13, emb_14, emb_15, emb_16,
           W0, b0, W1, b1, W2, b2, W3, b3, g0, beta0, g1, beta1, g2, beta2):
    embs = [emb_0, emb_1, emb_2, emb_3, emb_4, emb_5, emb_6, emb_7, emb_8,
            emb_9, emb_10, emb_11, emb_12, emb_13, emb_14, emb_15, emb_16]
    idx_t = (x + _OFFS[None, :]).T
    h = _sc_gather()(*embs, idx_t)
    r = lambda a: a.reshape(1, -1)
    out = _mlp(h, W0, r(b0), r(g0), r(beta0), W1, r(b1), r(g1), r(beta1),
               W2, r(b2), r(g2), r(beta2), W3, r(b3))
    return out


# speedup vs baseline: 1.0229x; 1.0229x over previous
import functools

import numpy as np
import jax
import jax.numpy as jnp
from jax import lax
from jax.experimental import pallas as pl
from jax.experimental.pallas import tpu as pltpu
from jax.experimental.pallas import tpu_sc as plsc

_BINS = (512, 128, 256, 256, 64, 256, 256, 16, 256, 64, 16, 128, 64, 128, 64, 512, 512)
_EMB = 16
_NF = 17
_BATCH = 4096
_EPS = 1e-5
_OFFS = np.concatenate([[0], np.cumsum(_BINS)[:-1]]).astype(np.int32)
_VOCAB = int(np.sum(_BINS))

_NC, _NS = 2, 16
_NW = _NC * _NS
_ROWS_PW = _BATCH // _NW
_WIDTH = _NF * _EMB
_LTILES = (_WIDTH + 127) // 128
_SLAB = _ROWS_PW * _LTILES * 8 * 128 // 8
_OUT_FLAT = _NW * _SLAB


def _gather_body(table_hbm, idx_hbm, h_hbm, table_v, idx_v, out_v, sem, sem2):
    wid = lax.axis_index("s") * _NC + lax.axis_index("c")

    tcopy = pltpu.async_copy(table_hbm, table_v, sem)
    icopies = [
        pltpu.async_copy(
            idx_hbm.at[pl.ds(i * _BATCH + wid * _ROWS_PW, _ROWS_PW)],
            idx_v.at[i], sem2)
        for i in range(_NF)
    ]
    tcopy.wait()
    for c in icopies:
        c.wait()

    iota = lax.iota(jnp.int32, 16)
    rowpart_c = (iota // 8) * 3072 + (iota % 8) * 128

    ones = jnp.ones((16,), jnp.int32)

    def step(g, _):
        rowpart = rowpart_c + g * 2 * 3072

        def fstep(i, _):
            idx = idx_v[i, pl.ds(g * 16, 16)]
            col0 = i * _EMB
            base = rowpart + (col0 // 128) * 1024 + lax.rem(col0, 128)
            csplat = jnp.zeros((16,), jnp.int32)
            for _c in range(_EMB):
                val = plsc.load_gather(table_v, [idx, csplat])
                plsc.store_scatter(out_v, [base], val)
                csplat = csplat + ones
                base = base + ones
            return _

        return lax.fori_loop(0, _NF, fstep, _)

    lax.fori_loop(0, _ROWS_PW // 16, step, None)

    pltpu.sync_copy(out_v, h_hbm.at[pl.ds(wid * _SLAB, _SLAB)])


@functools.lru_cache(maxsize=None)
def _sc_gather():
    return pl.kernel(
        _gather_body,
        out_type=jax.ShapeDtypeStruct((_OUT_FLAT,), jnp.float32),
        mesh=plsc.VectorSubcoreMesh(core_axis_name="c", subcore_axis_name="s",
                                    num_cores=_NC, num_subcores=_NS),
        scratch_types=[
            pltpu.VMEM((_VOCAB, _EMB), jnp.float32),
            pltpu.VMEM((_NF, _ROWS_PW), jnp.int32),
            pltpu.VMEM((_SLAB,), jnp.float32),
            pltpu.SemaphoreType.DMA,
            pltpu.SemaphoreType.DMA,
        ],
        compiler_params=pltpu.CompilerParams(use_tc_tiling_on_sc=False,
                                             needs_layout_passes=False),
    )


def _mlp_body(x4_ref, w0, b0, g0, be0, w1, b1, g1, be1, w2, b2, g2, be2,
              w3, b3, out_ref):
    x4 = x4_ref[:]
    parts = [x4[:, c, :, :].reshape(_BATCH, 128) for c in range(_LTILES)]
    h = jnp.concatenate(parts, axis=1)[:, :_WIDTH]

    def layer(h, w, b, g, be):
        h = jnp.dot(h, w[:], preferred_element_type=jnp.float32) + b[:]
        h = jnp.maximum(h, 0.0)
        m = jnp.mean(h, axis=0, keepdims=True)
        v = jnp.mean((h - m) ** 2, axis=0, keepdims=True)
        return (h - m) * (g[:] * lax.rsqrt(v + _EPS)) + be[:]

    h = layer(h, w0, b0, g0, be0)
    h = layer(h, w1, b1, g1, be1)
    h = layer(h, w2, b2, g2, be2)
    out_ref[:] = jnp.dot(h, w3[:], preferred_element_type=jnp.float32) + b3[:]


_mlp = pl.pallas_call(
    _mlp_body,
    out_shape=jax.ShapeDtypeStruct((_BATCH, 1), jnp.float32),
)


def kernel(x, emb_0, emb_1, emb_2, emb_3, emb_4, emb_5, emb_6, emb_7, emb_8,
           emb_9, emb_10, emb_11, emb_12, emb_13, emb_14, emb_15, emb_16,
           W0, b0, W1, b1, W2, b2, W3, b3, g0, beta0, g1, beta1, g2, beta2):
    embs = [emb_0, emb_1, emb_2, emb_3, emb_4, emb_5, emb_6, emb_7, emb_8,
            emb_9, emb_10, emb_11, emb_12, emb_13, emb_14, emb_15, emb_16]
    table = jnp.concatenate(embs, axis=0)
    idx_fm = (x + _OFFS[None, :]).T.reshape(-1)
    h_flat = _sc_gather()(table, idx_fm)
    x4 = h_flat.reshape(_BATCH // 8, _LTILES, 8, 128)
    r = lambda a: a.reshape(1, -1)
    out = _mlp(x4, W0, r(b0), r(g0), r(beta0), W1, r(b1), r(g1), r(beta1),
               W2, r(b2), r(g2), r(beta2), W3, r(b3))
    return out
